# Initial kernel scaffold; baseline (speedup 1.0000x reference)
import jax
import jax.numpy as jnp
from jax.experimental import pallas as pl


def kernel(x, edge_index, edge_weight, W1, W2, W_out, b_out):
    raise NotImplementedError("write your pallas kernel here")


# speedup vs baseline: 11.3325x; 11.3325x over previous
import functools

import jax
import jax.numpy as jnp
from jax import lax
from jax.experimental import pallas as pl
from jax.experimental.pallas import tpu as pltpu
from jax.experimental.pallas import tpu_sc as plsc

N_NODES = 10000
N_EDGES = 320000
D_IN = 128
H1 = 16
H2 = 32
D_OUT = 40

NC = 2
NS = 16
NW = NC * NS
CHUNK = 128
NCH = 79
EP = NW * NCH * CHUNK
N_PAD = 10112
ROWS_PER_TILE = N_PAD // NS


def _make_spmm(F):
    mesh = plsc.VectorSubcoreMesh(core_axis_name="c", subcore_axis_name="s")

    @functools.partial(
        pl.kernel,
        out_type=jax.ShapeDtypeStruct((NC, N_PAD, F), jnp.float32),
        mesh=mesh,
        scratch_types=[
            pltpu.VMEM((NCH, CHUNK), jnp.int32),
            pltpu.VMEM((NCH, CHUNK), jnp.int32),
            pltpu.VMEM((NCH, CHUNK), jnp.float32),
            pltpu.VMEM((CHUNK, F), jnp.float32),
            pltpu.VMEM((CHUNK, F), jnp.float32),
            pltpu.VMEM_SHARED((N_PAD, F), jnp.float32),
            pltpu.SemaphoreType.DMA,
        ],
        compiler_params=pltpu.CompilerParams(use_tc_tiling_on_sc=False),
    )
    def spmm(feat_hbm, src_hbm, dst_hbm, w_hbm, out_hbm,
             src_v, dst_v, w_v, rows_v, zero_v, acc_sh, sem):
        cid = lax.axis_index("c")
        sid = lax.axis_index("s")
        wid = sid * NC + cid

        def zrow(i, carry):
            for fb in range(F // 16):
                zero_v[i, pl.ds(fb * 16, 16)] = jnp.zeros((16,), jnp.float32)
            return carry
        lax.fori_loop(0, CHUNK, zrow, 0)
        zbase = sid * ROWS_PER_TILE
        off = 0
        while off < ROWS_PER_TILE:
            n = min(CHUNK, ROWS_PER_TILE - off)
            pltpu.sync_copy(zero_v.at[pl.ds(0, n)],
                            acc_sh.at[pl.ds(zbase + off, n)])
            off += n
        plsc.subcore_barrier()

        pltpu.sync_copy(src_hbm.at[wid], src_v)
        pltpu.sync_copy(dst_hbm.at[wid], dst_v)
        pltpu.sync_copy(w_hbm.at[wid], w_v)

        def chunk_body(j, carry):
            pltpu.async_copy(feat_hbm.at[src_v.at[j]], rows_v, sem).wait()

            def scale(g, c2):
                wv = w_v[j, pl.ds(g * 16, 16)]
                for k in range(16):
                    s = wv[k]
                    e = g * 16 + k
                    for fb in range(F // 16):
                        sl = pl.ds(fb * 16, 16)
                        rows_v[e, sl] = rows_v[e, sl] * s
                return c2
            lax.fori_loop(0, CHUNK // 16, scale, 0)

            pltpu.sync_copy(rows_v, acc_sh.at[dst_v.at[j]], add=True)
            return carry
        lax.fori_loop(0, NCH, chunk_body, 0)

        plsc.subcore_barrier()
        base = sid * ROWS_PER_TILE
        pltpu.sync_copy(acc_sh.at[pl.ds(base, ROWS_PER_TILE)],
                        out_hbm.at[cid, pl.ds(base, ROWS_PER_TILE)])

    return spmm


_spmm_h1 = _make_spmm(H1)
_spmm_h2 = _make_spmm(H2)


def _tc_in(x, W1):
    def body(x_ref, w_ref, o_ref):
        o_ref[...] = jnp.dot(x_ref[...], w_ref[...],
                             preferred_element_type=jnp.float32)
    return pl.pallas_call(
        body, out_shape=jax.ShapeDtypeStruct((N_NODES, H1), jnp.float32),
    )(x, W1)


def _tc_mid(parts, W2):
    def body(p_ref, w_ref, o_ref):
        h = jnp.maximum(p_ref[0] + p_ref[1], 0.0)
        o_ref[...] = jnp.dot(h, w_ref[...], preferred_element_type=jnp.float32)
    return pl.pallas_call(
        body, out_shape=jax.ShapeDtypeStruct((N_PAD, H2), jnp.float32),
    )(parts, W2)


def _tc_out(parts, W_out, b_out2d):
    def body(p_ref, w_ref, b_ref, o_ref):
        h = jnp.maximum(p_ref[0] + p_ref[1], 0.0)
        o_ref[...] = (jnp.dot(h, w_ref[...], preferred_element_type=jnp.float32)
                      + b_ref[...])
    return pl.pallas_call(
        body, out_shape=jax.ShapeDtypeStruct((N_PAD, D_OUT), jnp.float32),
    )(parts, W_out, b_out2d)


def kernel(x, edge_index, edge_weight, W1, W2, W_out, b_out):
    pad = EP - N_EDGES
    src = jnp.concatenate([edge_index[0],
                           jnp.zeros((pad,), jnp.int32)]).reshape(NW, NCH, CHUNK)
    dst = jnp.concatenate([edge_index[1],
                           jnp.full((pad,), N_NODES, jnp.int32)]).reshape(NW, NCH, CHUNK)
    w = jnp.concatenate([edge_weight,
                         jnp.zeros((pad,), jnp.float32)]).reshape(NW, NCH, CHUNK)

    xw = _tc_in(x, W1)
    p1 = _spmm_h1(xw, src, dst, w)
    hw = _tc_mid(p1, W2)
    p2 = _spmm_h2(hw, src, dst, w)
    out = _tc_out(p2, W_out, b_out.reshape(1, D_OUT))
    return out[:N_NODES]


# speedup vs baseline: 12.8654x; 1.1353x over previous
import functools

import jax
import jax.numpy as jnp
from jax import lax
from jax.experimental import pallas as pl
from jax.experimental.pallas import tpu as pltpu
from jax.experimental.pallas import tpu_sc as plsc

N_NODES = 10000
N_EDGES = 320000
D_IN = 128
H1 = 16
H2 = 32
D_OUT = 40

NC = 2
NS = 16
NW = NC * NS
CHUNK = 128
NCH = 80
EP = NW * NCH * CHUNK
N_PAD = 10112
ROWS_PER_TILE = N_PAD // NS


def _make_spmm(F):
    mesh = plsc.VectorSubcoreMesh(core_axis_name="c", subcore_axis_name="s")

    @functools.partial(
        pl.kernel,
        out_type=jax.ShapeDtypeStruct((NC, N_PAD, F), jnp.float32),
        mesh=mesh,
        scratch_types=[
            pltpu.VMEM((NCH, CHUNK), jnp.int32),
            pltpu.VMEM((NCH, CHUNK), jnp.int32),
            pltpu.VMEM((NCH, CHUNK), jnp.float32),
            pltpu.VMEM((2, CHUNK, F), jnp.float32),
            pltpu.VMEM((CHUNK, F), jnp.float32),
            pltpu.VMEM_SHARED((N_PAD, F), jnp.float32),
            pltpu.SemaphoreType.DMA,
            pltpu.SemaphoreType.DMA,
        ],
        compiler_params=pltpu.CompilerParams(use_tc_tiling_on_sc=False),
    )
    def spmm(feat_hbm, src_hbm, dst_hbm, w_hbm, out_hbm,
             src_v, dst_v, w_v, rows_v, zero_v, acc_sh, gsem0, gsem1):
        gsems = (gsem0, gsem1)
        cid = lax.axis_index("c")
        sid = lax.axis_index("s")
        wid = sid * NC + cid

        def zrow(i, carry):
            for fb in range(F // 16):
                zero_v[i, pl.ds(fb * 16, 16)] = jnp.zeros((16,), jnp.float32)
            return carry
        lax.fori_loop(0, CHUNK, zrow, 0)
        zbase = sid * ROWS_PER_TILE
        off = 0
        while off < ROWS_PER_TILE:
            n = min(CHUNK, ROWS_PER_TILE - off)
            pltpu.sync_copy(zero_v.at[pl.ds(0, n)],
                            acc_sh.at[pl.ds(zbase + off, n)])
            off += n
        plsc.subcore_barrier()

        pltpu.sync_copy(src_hbm.at[wid], src_v)
        pltpu.sync_copy(dst_hbm.at[wid], dst_v)
        pltpu.sync_copy(w_hbm.at[wid], w_v)

        for b in range(2):
            pltpu.async_copy(feat_hbm.at[src_v.at[b]], rows_v.at[b], gsems[b])

        def chunk_pair(i, carry):
            for b in range(2):
                j = 2 * i + b
                buf = rows_v.at[b]
                pltpu.make_async_copy(feat_hbm.at[src_v.at[j]], buf,
                                      gsems[b]).wait()

                for g in range(CHUNK // 16):
                    wv = w_v[j, pl.ds(g * 16, 16)]
                    for k in range(16):
                        s = wv[k]
                        e = g * 16 + k
                        for fb in range(F // 16):
                            sl = pl.ds(fb * 16, 16)
                            buf[e, sl] = buf[e, sl] * s

                pltpu.sync_copy(buf, acc_sh.at[dst_v.at[j]], add=True)

                @pl.when(j + 2 < NCH)
                def _():
                    pltpu.async_copy(feat_hbm.at[src_v.at[j + 2]], buf,
                                     gsems[b])
            return carry
        lax.fori_loop(0, NCH // 2, chunk_pair, 0)

        plsc.subcore_barrier()
        base = sid * ROWS_PER_TILE
        pltpu.sync_copy(acc_sh.at[pl.ds(base, ROWS_PER_TILE)],
                        out_hbm.at[cid, pl.ds(base, ROWS_PER_TILE)])

    return spmm


_spmm_h1 = _make_spmm(H1)
_spmm_h2 = _make_spmm(H2)


def _tc_in(x, W1):
    def body(x_ref, w_ref, o_ref):
        o_ref[...] = jnp.dot(x_ref[...], w_ref[...],
                             preferred_element_type=jnp.float32)
    return pl.pallas_call(
        body, out_shape=jax.ShapeDtypeStruct((N_NODES, H1), jnp.float32),
    )(x, W1)


def _tc_mid(parts, W2):
    def body(p_ref, w_ref, o_ref):
        h = jnp.maximum(p_ref[0] + p_ref[1], 0.0)
        o_ref[...] = jnp.dot(h, w_ref[...], preferred_element_type=jnp.float32)
    return pl.pallas_call(
        body, out_shape=jax.ShapeDtypeStruct((N_PAD, H2), jnp.float32),
    )(parts, W2)


def _tc_out(parts, W_out, b_out2d):
    def body(p_ref, w_ref, b_ref, o_ref):
        h = jnp.maximum(p_ref[0] + p_ref[1], 0.0)
        o_ref[...] = (jnp.dot(h, w_ref[...], preferred_element_type=jnp.float32)
                      + b_ref[...])
    return pl.pallas_call(
        body, out_shape=jax.ShapeDtypeStruct((N_PAD, D_OUT), jnp.float32),
    )(parts, W_out, b_out2d)


def kernel(x, edge_index, edge_weight, W1, W2, W_out, b_out):
    pad = EP - N_EDGES
    src = jnp.concatenate([edge_index[0],
                           jnp.zeros((pad,), jnp.int32)]).reshape(NW, NCH, CHUNK)
    dst = jnp.concatenate([edge_index[1],
                           jnp.full((pad,), N_NODES, jnp.int32)]).reshape(NW, NCH, CHUNK)
    w = jnp.concatenate([edge_weight,
                         jnp.zeros((pad,), jnp.float32)]).reshape(NW, NCH, CHUNK)

    xw = _tc_in(x, W1)
    p1 = _spmm_h1(xw, src, dst, w)
    hw = _tc_mid(p1, W2)
    p2 = _spmm_h2(hw, src, dst, w)
    out = _tc_out(p2, W_out, b_out.reshape(1, D_OUT))
    return out[:N_NODES]


# speedup vs baseline: 14.2450x; 1.1072x over previous
import functools

import jax
import jax.numpy as jnp
from jax import lax
from jax.experimental import pallas as pl
from jax.experimental.pallas import tpu as pltpu
from jax.experimental.pallas import tpu_sc as plsc

N_NODES = 10000
N_EDGES = 320000
D_IN = 128
H1 = 16
H2 = 32
D_OUT = 40

NC = 2
NS = 16
NW = NC * NS
CHUNK = 128
NCH = 80
EP = NW * NCH * CHUNK
N_PAD = 10112
ROWS_PER_TILE = N_PAD // NS


def _make_spmm(F):
    mesh = plsc.VectorSubcoreMesh(core_axis_name="c", subcore_axis_name="s")

    @functools.partial(
        pl.kernel,
        out_type=jax.ShapeDtypeStruct((NC, N_PAD, F), jnp.float32),
        mesh=mesh,
        scratch_types=[
            pltpu.VMEM((NCH, CHUNK), jnp.int32),
            pltpu.VMEM((NCH, CHUNK), jnp.int32),
            pltpu.VMEM((NCH, CHUNK), jnp.float32),
            pltpu.VMEM((4, CHUNK, F), jnp.float32),
            pltpu.VMEM((CHUNK, F), jnp.float32),
            pltpu.VMEM_SHARED((N_PAD, F), jnp.float32),
            [pltpu.SemaphoreType.DMA] * 4,
            [pltpu.SemaphoreType.DMA] * 4,
            pltpu.SemaphoreType.DMA,
        ],
        compiler_params=pltpu.CompilerParams(use_tc_tiling_on_sc=False),
    )
    def spmm(feat_hbm, src_hbm, dst_hbm, w_hbm, out_hbm,
             src_v, dst_v, w_v, rows_v, zero_v, acc_sh, gsems, ssems, esem):
        cid = lax.axis_index("c")
        sid = lax.axis_index("s")
        wid = sid * NC + cid

        pltpu.async_copy(src_hbm.at[wid], src_v, esem)
        pltpu.async_copy(dst_hbm.at[wid], dst_v, esem)
        pltpu.async_copy(w_hbm.at[wid], w_v, esem)

        def zrow(i, carry):
            for fb in range(F // 16):
                zero_v[i, pl.ds(fb * 16, 16)] = jnp.zeros((16,), jnp.float32)
            return carry
        lax.fori_loop(0, CHUNK, zrow, 0)
        zbase = sid * ROWS_PER_TILE
        off = 0
        while off < ROWS_PER_TILE:
            n = min(CHUNK, ROWS_PER_TILE - off)
            pltpu.sync_copy(zero_v.at[pl.ds(0, n)],
                            acc_sh.at[pl.ds(zbase + off, n)])
            off += n

        pltpu.make_async_copy(src_hbm.at[wid], src_v, esem).wait()
        pltpu.make_async_copy(dst_hbm.at[wid], dst_v, esem).wait()
        pltpu.make_async_copy(w_hbm.at[wid], w_v, esem).wait()

        for b in range(2):
            pltpu.async_copy(feat_hbm.at[src_v.at[b]], rows_v.at[b], gsems[b])
        plsc.subcore_barrier()

        def gather_wait(j, b):
            pltpu.make_async_copy(feat_hbm.at[src_v.at[j]], rows_v.at[b],
                                  gsems[b]).wait()

        def scatter_wait(j, b):
            pltpu.make_async_copy(rows_v.at[b], acc_sh.at[dst_v.at[j]],
                                  ssems[b]).wait()

        def round4(i, carry):
            for b in range(4):
                j = 4 * i + b
                buf = rows_v.at[b]
                gather_wait(j, b)

                for g in range(CHUNK // 16):
                    wv = w_v[j, pl.ds(g * 16, 16)]
                    for k in range(16):
                        s = wv[k]
                        e = g * 16 + k
                        for fb in range(F // 16):
                            sl = pl.ds(fb * 16, 16)
                            buf[e, sl] = buf[e, sl] * s

                pltpu.async_copy(buf, acc_sh.at[dst_v.at[j]], ssems[b],
                                 add=True)

                jp = j + 2
                bp = (b + 2) % 4

                @pl.when(jnp.logical_and(jp >= 4, jp < NCH))
                def _():
                    scatter_wait(jp - 4, bp)
                    pltpu.async_copy(feat_hbm.at[src_v.at[jp]],
                                     rows_v.at[bp], gsems[bp])

                @pl.when(jnp.logical_and(jp < 4, jp < NCH))
                def _():
                    pltpu.async_copy(feat_hbm.at[src_v.at[jp]],
                                     rows_v.at[bp], gsems[bp])
            return carry
        lax.fori_loop(0, NCH // 4, round4, 0)

        scatter_wait(NCH - 2, (NCH - 2) % 4)
        scatter_wait(NCH - 1, (NCH - 1) % 4)

        plsc.subcore_barrier()
        base = sid * ROWS_PER_TILE
        pltpu.sync_copy(acc_sh.at[pl.ds(base, ROWS_PER_TILE)],
                        out_hbm.at[cid, pl.ds(base, ROWS_PER_TILE)])

    return spmm


_spmm_h1 = _make_spmm(H1)
_spmm_h2 = _make_spmm(H2)


def _tc_in(x, W1):
    def body(x_ref, w_ref, o_ref):
        o_ref[...] = jnp.dot(x_ref[...], w_ref[...],
                             preferred_element_type=jnp.float32)
    return pl.pallas_call(
        body, out_shape=jax.ShapeDtypeStruct((N_NODES, H1), jnp.float32),
    )(x, W1)


def _tc_mid(parts, W2):
    def body(p_ref, w_ref, o_ref):
        h = jnp.maximum(p_ref[0] + p_ref[1], 0.0)
        o_ref[...] = jnp.dot(h, w_ref[...], preferred_element_type=jnp.float32)
    return pl.pallas_call(
        body, out_shape=jax.ShapeDtypeStruct((N_PAD, H2), jnp.float32),
    )(parts, W2)


def _tc_out(parts, W_out, b_out2d):
    def body(p_ref, w_ref, b_ref, o_ref):
        h = jnp.maximum(p_ref[0, :N_NODES] + p_ref[1, :N_NODES], 0.0)
        o_ref[...] = (jnp.dot(h, w_ref[...], preferred_element_type=jnp.float32)
                      + b_ref[...])
    return pl.pallas_call(
        body, out_shape=jax.ShapeDtypeStruct((N_NODES, D_OUT), jnp.float32),
    )(parts, W_out, b_out2d)


def kernel(x, edge_index, edge_weight, W1, W2, W_out, b_out):
    pad = EP - N_EDGES
    src = jnp.concatenate([edge_index[0],
                           jnp.zeros((pad,), jnp.int32)]).reshape(NW, NCH, CHUNK)
    dst = jnp.concatenate([edge_index[1],
                           jnp.full((pad,), N_NODES, jnp.int32)]).reshape(NW, NCH, CHUNK)
    w = jnp.concatenate([edge_weight,
                         jnp.zeros((pad,), jnp.float32)]).reshape(NW, NCH, CHUNK)

    xw = _tc_in(x, W1)
    p1 = _spmm_h1(xw, src, dst, w)
    hw = _tc_mid(p1, W2)
    p2 = _spmm_h2(hw, src, dst, w)
    return _tc_out(p2, W_out, b_out.reshape(1, D_OUT))
